# baseline (speedup 1.0000x reference)
import functools

import jax
import jax.numpy as jnp
from jax import lax
from jax.experimental import pallas as pl
from jax.experimental.pallas import tpu as pltpu
from jax.experimental.pallas import tpu_sc as plsc

_B, _C, _H, _W = 4, 96, 384, 384
_P = _B * _C
_HW = _H * _W
_G = _HW // 16
_K = 4096
_NC, _NS = 2, 16
_NW = _NC * _NS
_PPW = _P // _NW
_CH = 128
_BATCH = 2048
_NB = _PPW * _K // _BATCH
_NCH = _BATCH // _CH
_L = 16


def _sc_gather(x3, locf):
    mesh = plsc.VectorSubcoreMesh(
        core_axis_name="c", subcore_axis_name="s",
        num_cores=_NC, num_subcores=_NS)

    @functools.partial(
        pl.kernel,
        out_type=jax.ShapeDtypeStruct((_P * _K,), jnp.float32),
        mesh=mesh,
        compiler_params=pltpu.CompilerParams(
            needs_layout_passes=False, use_tc_tiling_on_sc=False),
        scratch_types=[
            pltpu.VMEM((2 * _K,), jnp.int32),
            pltpu.VMEM((_K,), jnp.int32),
            pltpu.VMEM((_K,), jnp.int32),
            pltpu.VMEM((2, _BATCH, _L), jnp.float32),
            pltpu.VMEM((_BATCH,), jnp.float32),
            pltpu.SemaphoreType.DMA,
            pltpu.SemaphoreType.DMA,
        ],
    )
    def k(x_hbm, loc_hbm, out_hbm, loc_v, gid_v, off_v, buf_v, outv_v,
          sem0, sem1):
        wid = lax.axis_index("s") * _NC + lax.axis_index("c")
        base_p = wid * _PPW
        sems = (sem0, sem1)

        pltpu.sync_copy(loc_hbm, loc_v)

        def idx_body(i, _):
            s = i * _L
            lin = loc_v[pl.ds(s, _L)] * _W + loc_v[pl.ds(_K + s, _L)]
            gid_v[pl.ds(s, _L)] = lax.shift_right_logical(lin, 4)
            off_v[pl.ds(s, _L)] = lax.bitwise_and(lin, 15)
            return 0
        lax.fori_loop(0, _K // _L, idx_body, 0)

        def fire(b, slot):
            p = base_p + b // 2
            kbase = (b % 2) * _BATCH
            for c in range(_NCH):
                src = x_hbm.at[p].at[gid_v.at[pl.ds(kbase + c * _CH, _CH)]]
                dst = buf_v.at[slot].at[pl.ds(c * _CH, _CH)]
                pltpu.async_copy(src, dst, sems[slot])

        def drain(slot):
            pltpu.make_async_copy(
                x_hbm.at[0].at[pl.ds(0, _BATCH)], buf_v.at[slot],
                sems[slot]).wait()

        lane_i = lax.iota(jnp.int32, _L)

        def extract_and_out(b, slot):
            p = base_p + b // 2
            kbase = (b % 2) * _BATCH

            def ext_body(e, _):
                row = e * _L + lane_i
                col = off_v[pl.ds(kbase + e * _L, _L)]
                outv_v[pl.ds(e * _L, _L)] = plsc.load_gather(
                    buf_v.at[slot], [row, col])
                return 0
            lax.fori_loop(0, _BATCH // _L, ext_body, 0)
            pltpu.sync_copy(outv_v, out_hbm.at[pl.ds(p * _K + kbase, _BATCH)])

        fire(0, 0)

        def round_body(r, _):
            b0 = 2 * r
            fire(b0 + 1, 1)
            drain(0)
            extract_and_out(b0, 0)

            @pl.when(b0 + 2 < _NB)
            def _():
                fire(b0 + 2, 0)
            drain(1)
            extract_and_out(b0 + 1, 1)
            return 0
        lax.fori_loop(0, _NB // 2, round_body, 0)

    return k(x3, locf)


def kernel(x, loc):
    x3 = x.reshape(_P, _G, _L)
    locf = loc.reshape(2 * _K).astype(jnp.int32)
    return _sc_gather(x3, locf).reshape(_B, _C, _K)


# speedup vs baseline: 2.1285x; 2.1285x over previous
import functools

import jax
import jax.numpy as jnp
from jax import lax
from jax.experimental import pallas as pl
from jax.experimental.pallas import tpu as pltpu
from jax.experimental.pallas import tpu_sc as plsc

_B, _C, _H, _W = 4, 96, 384, 384
_P = _B * _C
_K = 4096
_NC, _NS = 2, 16
_NW = _NC * _NS
_PPW = _P // _NW
_SR = 64
_NSL = _H // _SR
_STEPS = _PPW * _NSL
_LROW = _K + 16
_L = 16


def _sc_gather(x, locf):
    mesh = plsc.VectorSubcoreMesh(
        core_axis_name="c", subcore_axis_name="s",
        num_cores=_NC, num_subcores=_NS)

    @functools.partial(
        pl.kernel,
        out_type=jax.ShapeDtypeStruct((_P * _K,), jnp.float32),
        mesh=mesh,
        compiler_params=pltpu.CompilerParams(
            needs_layout_passes=False, use_tc_tiling_on_sc=True),
        scratch_types=[
            pltpu.VMEM((2 * _K,), jnp.int32),
            pltpu.VMEM((_NSL * _LROW,), jnp.int32),
            pltpu.VMEM((2, _SR, _W), jnp.float32),
            pltpu.VMEM((2 * _K,), jnp.float32),
            pltpu.SMEM((8,), jnp.int32),
            pltpu.SemaphoreType.DMA,
            pltpu.SemaphoreType.DMA,
            pltpu.SemaphoreType.DMA,
        ],
    )
    def k(x_hbm, loc_hbm, out_hbm, loc_v, lists_v, buf_v, outv_v, cnt_s,
          sem0, sem1, semo):
        wid = lax.axis_index("s") * _NC + lax.axis_index("c")
        base_p = wid * _PPW
        sems = (sem0, sem1)
        lane_i = lax.iota(jnp.int32, _L)

        pltpu.sync_copy(loc_hbm, loc_v)

        for q in range(_NSL):
            def build(i, ptr):
                s = i * _L
                r = loc_v[pl.ds(s, _L)]
                c = loc_v[pl.ds(_K + s, _L)]
                w = lax.shift_left(s + lane_i, 18) | lax.shift_left(r, 9) | c
                m = lax.shift_right_logical(r, 6) == q
                plsc.store_compressed(
                    lists_v.at[pl.ds(q * _LROW + ptr, _L)], w, mask=m)
                return ptr + jnp.sum(jnp.where(m, 1, 0))
            cnt_s[q] = lax.fori_loop(0, _K // _L, build, 0)

        def fire(step, slot):
            j = step // _NSL
            q = step % _NSL
            p = base_p + j
            src = x_hbm.at[p // _C, p % _C, pl.ds(q * _SR, _SR), :]
            pltpu.async_copy(src, buf_v.at[slot], sems[slot])

        def drain(slot):
            pltpu.make_async_copy(
                x_hbm.at[0, 0, pl.ds(0, _SR), :], buf_v.at[slot],
                sems[slot]).wait()

        def extract(step, slot):
            j = step // _NSL
            q = step % _NSL
            p = base_p + j
            nq = cnt_s[q]
            ovec = outv_v.at[pl.ds((j % 2) * _K, _K)]

            @pl.when((q == 0) & (j >= 2))
            def _():
                pltpu.make_async_copy(
                    outv_v.at[pl.ds(0, _K)], out_hbm.at[pl.ds(0, _K)],
                    semo).wait()

            def ext_body(i, _):
                base = i * _L
                w = lists_v[pl.ds(q * _LROW + base, _L)]
                kk = lax.shift_right_logical(w, 18)
                rloc = lax.bitwise_and(lax.shift_right_logical(w, 9), 63)
                cc = lax.bitwise_and(w, 511)
                m = (base + lane_i) < nq
                vals = plsc.load_gather(buf_v.at[slot], [rloc, cc], mask=m)
                plsc.store_scatter(ovec, [kk], vals, mask=m)
                return 0
            lax.fori_loop(0, (nq + _L - 1) // _L, ext_body, 0)

            @pl.when(q == _NSL - 1)
            def _():
                pltpu.async_copy(ovec, out_hbm.at[pl.ds(p * _K, _K)], semo)

        fire(0, 0)

        def round_body(r, _):
            s0 = 2 * r
            fire(s0 + 1, 1)
            drain(0)
            extract(s0, 0)

            @pl.when(s0 + 2 < _STEPS)
            def _():
                fire(s0 + 2, 0)
            drain(1)
            extract(s0 + 1, 1)
            return 0
        lax.fori_loop(0, _STEPS // 2, round_body, 0)

        pltpu.make_async_copy(
            outv_v.at[pl.ds(0, 2 * _K)], out_hbm.at[pl.ds(0, 2 * _K)],
            semo).wait()

    return k(x, locf)


def kernel(x, loc):
    locf = loc.reshape(2 * _K).astype(jnp.int32)
    return _sc_gather(x, locf).reshape(_B, _C, _K)


# speedup vs baseline: 2.4572x; 1.1544x over previous
import functools

import jax
import jax.numpy as jnp
from jax import lax
from jax.experimental import pallas as pl
from jax.experimental.pallas import tpu as pltpu
from jax.experimental.pallas import tpu_sc as plsc

_B, _C, _H, _W = 4, 96, 384, 384
_P = _B * _C
_K = 4096
_NC, _NS = 2, 16
_NW = _NC * _NS
_PPW = _P // _NW
_SR = 128
_NSL = _H // _SR
_STEPS = _PPW * _NSL
_LROW = _K + 16
_L = 16


def _sc_gather(x, locf):
    mesh = plsc.VectorSubcoreMesh(
        core_axis_name="c", subcore_axis_name="s",
        num_cores=_NC, num_subcores=_NS)

    @functools.partial(
        pl.kernel,
        out_type=jax.ShapeDtypeStruct((_B, _C, _K), jnp.float32),
        mesh=mesh,
        compiler_params=pltpu.CompilerParams(
            needs_layout_passes=False, use_tc_tiling_on_sc=True),
        scratch_types=[
            pltpu.VMEM((2 * _K,), jnp.int32),
            pltpu.VMEM((_NSL * _LROW,), jnp.int32),
            pltpu.VMEM((2, _SR, _W), jnp.float32),
            pltpu.VMEM((2 * _K,), jnp.float32),
            pltpu.SMEM((8,), jnp.int32),
            pltpu.SemaphoreType.DMA,
            pltpu.SemaphoreType.DMA,
            pltpu.SemaphoreType.DMA,
        ],
    )
    def k(x_hbm, loc_hbm, out_hbm, loc_v, lists_v, buf_v, outv_v, cnt_s,
          sem0, sem1, semo):
        wid = lax.axis_index("s") * _NC + lax.axis_index("c")
        base_p = wid * _PPW
        sems = (sem0, sem1)
        lane_i = lax.iota(jnp.int32, _L)

        pltpu.sync_copy(loc_hbm, loc_v)

        for q in range(_NSL):
            def build(i, ptr):
                s = i * _L
                r = loc_v[pl.ds(s, _L)]
                c = loc_v[pl.ds(_K + s, _L)]
                rloc = lax.bitwise_and(r, _SR - 1)
                w = (lax.shift_left(s + lane_i, 16)
                     | lax.shift_left(rloc, 9) | c)
                m = lax.shift_right_logical(r, 7) == q
                plsc.store_compressed(
                    lists_v.at[pl.ds(q * _LROW + ptr, _L)], w, mask=m)
                return ptr + jnp.sum(jnp.where(m, 1, 0))
            cnt_s[q] = lax.fori_loop(0, _K // _L, build, 0)

        def fire(step, slot):
            j = step // _NSL
            q = step % _NSL
            p = base_p + j
            src = x_hbm.at[p // _C, p % _C, pl.ds(q * _SR, _SR), :]
            pltpu.async_copy(src, buf_v.at[slot], sems[slot])

        def drain(slot):
            pltpu.make_async_copy(
                x_hbm.at[0, 0, pl.ds(0, _SR), :], buf_v.at[slot],
                sems[slot]).wait()

        def extract(step, slot):
            j = step // _NSL
            q = step % _NSL
            p = base_p + j
            nq = cnt_s[q]
            ovec = outv_v.at[pl.ds((j % 2) * _K, _K)]

            @pl.when((q == 0) & (j >= 2))
            def _():
                pltpu.make_async_copy(
                    outv_v.at[pl.ds(0, _K)], out_hbm.at[0, 0, :],
                    semo).wait()

            def ext_body(i, _):
                base = i * _L
                w = lists_v[pl.ds(q * _LROW + base, _L)]
                kk = lax.shift_right_logical(w, 16)
                rloc = lax.bitwise_and(
                    lax.shift_right_logical(w, 9), _SR - 1)
                cc = lax.bitwise_and(w, 511)
                m = (base + lane_i) < nq
                vals = plsc.load_gather(buf_v.at[slot], [rloc, cc], mask=m)
                plsc.store_scatter(ovec, [kk], vals, mask=m)
                return 0
            lax.fori_loop(0, (nq + _L - 1) // _L, ext_body, 0)

            @pl.when(q == _NSL - 1)
            def _():
                pltpu.async_copy(ovec, out_hbm.at[p // _C, p % _C, :], semo)

        fire(0, 0)

        def round_body(r, _):
            s0 = 2 * r
            fire(s0 + 1, 1)
            drain(0)
            extract(s0, 0)

            @pl.when(s0 + 2 < _STEPS)
            def _():
                fire(s0 + 2, 0)
            drain(1)
            extract(s0 + 1, 1)
            return 0
        lax.fori_loop(0, _STEPS // 2, round_body, 0)

        pltpu.make_async_copy(
            outv_v.at[pl.ds(0, _K)], out_hbm.at[0, 0, :], semo).wait()
        pltpu.make_async_copy(
            outv_v.at[pl.ds(0, _K)], out_hbm.at[0, 0, :], semo).wait()

    return k(x, locf)


def kernel(x, loc):
    locf = loc.reshape(2 * _K).astype(jnp.int32)
    return _sc_gather(x, locf)


# speedup vs baseline: 2.5400x; 1.0337x over previous
import functools

import jax
import jax.numpy as jnp
from jax import lax
from jax.experimental import pallas as pl
from jax.experimental.pallas import tpu as pltpu
from jax.experimental.pallas import tpu_sc as plsc

_B, _C, _H, _W = 4, 96, 384, 384
_P = _B * _C
_K = 4096
_NC, _NS = 2, 16
_NW = _NC * _NS
_PPW = _P // _NW
_SR = 128
_NSL = _H // _SR
_STEPS = _PPW * _NSL
_LROW = _K + 16
_L = 16


def _sc_gather(x, locf):
    mesh = plsc.VectorSubcoreMesh(
        core_axis_name="c", subcore_axis_name="s",
        num_cores=_NC, num_subcores=_NS)

    @functools.partial(
        pl.kernel,
        out_type=jax.ShapeDtypeStruct((_B, _C, _K), jnp.float32),
        mesh=mesh,
        compiler_params=pltpu.CompilerParams(
            needs_layout_passes=False, use_tc_tiling_on_sc=True),
        scratch_types=[
            pltpu.VMEM((2 * _K,), jnp.int32),
            pltpu.VMEM((_NSL * _LROW,), jnp.int32),
            pltpu.VMEM((2, _SR, _W), jnp.float32),
            pltpu.VMEM((2 * _K,), jnp.float32),
            pltpu.SMEM((8,), jnp.int32),
            pltpu.SemaphoreType.DMA,
            pltpu.SemaphoreType.DMA,
            pltpu.SemaphoreType.DMA,
        ],
    )
    def k(x_hbm, loc_hbm, out_hbm, loc_v, lists_v, buf_v, outv_v, cnt_s,
          sem0, sem1, semo):
        wid = lax.axis_index("s") * _NC + lax.axis_index("c")
        base_p = wid * _PPW
        sems = (sem0, sem1)
        lane_i = lax.iota(jnp.int32, _L)

        def fire(step, slot):
            j = step // _NSL
            q = step % _NSL
            p = base_p + j
            src = x_hbm.at[p // _C, p % _C, pl.ds(q * _SR, _SR), :]
            pltpu.async_copy(src, buf_v.at[slot], sems[slot])

        pltpu.sync_copy(loc_hbm, loc_v)
        fire(0, 0)
        fire(1, 1)

        for q in range(_NSL):
            def build(i, ptr):
                s = i * _L
                r = loc_v[pl.ds(s, _L)]
                c = loc_v[pl.ds(_K + s, _L)]
                rloc = lax.bitwise_and(r, _SR - 1)
                w = (lax.shift_left(s + lane_i, 16)
                     | lax.shift_left(rloc, 9) | c)
                m = lax.shift_right_logical(r, 7) == q
                plsc.store_compressed(
                    lists_v.at[pl.ds(q * _LROW + ptr, _L)], w, mask=m)
                return ptr + jnp.sum(jnp.where(m, 1, 0))
            cnt_s[q] = lax.fori_loop(0, _K // _L, build, 0)

        def drain(slot):
            pltpu.make_async_copy(
                x_hbm.at[0, 0, pl.ds(0, _SR), :], buf_v.at[slot],
                sems[slot]).wait()

        def extract(step, slot):
            j = step // _NSL
            q = step % _NSL
            p = base_p + j
            nq = cnt_s[q]
            ovec = outv_v.at[pl.ds((j % 2) * _K, _K)]

            @pl.when((q == 0) & (j >= 2))
            def _():
                pltpu.make_async_copy(
                    outv_v.at[pl.ds(0, _K)], out_hbm.at[0, 0, :],
                    semo).wait()

            def unpack(base):
                w = lists_v[pl.ds(q * _LROW + base, _L)]
                kk = lax.shift_right_logical(w, 16)
                rloc = lax.bitwise_and(
                    lax.shift_right_logical(w, 9), _SR - 1)
                cc = lax.bitwise_and(w, 511)
                return kk, rloc, cc

            def ext_body(i, _):
                kk, rloc, cc = unpack(i * _L)
                vals = plsc.load_gather(buf_v.at[slot], [rloc, cc])
                plsc.store_scatter(ovec, [kk], vals)
                return 0
            nfull = lax.shift_right_logical(nq, 4)
            lax.fori_loop(0, nfull, ext_body, 0)

            @pl.when(lax.bitwise_and(nq, _L - 1) != 0)
            def _():
                base = nfull * _L
                kk, rloc, cc = unpack(base)
                m = (base + lane_i) < nq
                vals = plsc.load_gather(buf_v.at[slot], [rloc, cc], mask=m)
                plsc.store_scatter(ovec, [kk], vals, mask=m)

            @pl.when(q == _NSL - 1)
            def _():
                pltpu.async_copy(ovec, out_hbm.at[p // _C, p % _C, :], semo)

        def round_body(r, _):
            s0 = 2 * r

            @pl.when(s0 > 0)
            def _():
                fire(s0 + 1, 1)
            drain(0)
            extract(s0, 0)

            @pl.when(s0 + 2 < _STEPS)
            def _():
                fire(s0 + 2, 0)
            drain(1)
            extract(s0 + 1, 1)
            return 0
        lax.fori_loop(0, _STEPS // 2, round_body, 0)

        pltpu.make_async_copy(
            outv_v.at[pl.ds(0, _K)], out_hbm.at[0, 0, :], semo).wait()
        pltpu.make_async_copy(
            outv_v.at[pl.ds(0, _K)], out_hbm.at[0, 0, :], semo).wait()

    return k(x, locf)


def kernel(x, loc):
    locf = loc.reshape(2 * _K).astype(jnp.int32)
    return _sc_gather(x, locf)


# speedup vs baseline: 2.5400x; 1.0000x over previous
import functools

import jax
import jax.numpy as jnp
from jax import lax
from jax.experimental import pallas as pl
from jax.experimental.pallas import tpu as pltpu
from jax.experimental.pallas import tpu_sc as plsc

_B, _C, _H, _W = 4, 96, 384, 384
_P = _B * _C
_K = 4096
_NC, _NS = 2, 16
_NW = _NC * _NS
_PPW = _P // _NW
_SR = 128
_NSL = _H // _SR
_STEPS = _PPW * _NSL
_LROW = _K + 16
_L = 16


def _sc_gather(x, locf):
    mesh = plsc.VectorSubcoreMesh(
        core_axis_name="c", subcore_axis_name="s",
        num_cores=_NC, num_subcores=_NS)

    @functools.partial(
        pl.kernel,
        out_type=jax.ShapeDtypeStruct((_B, _C, _K), jnp.float32),
        mesh=mesh,
        compiler_params=pltpu.CompilerParams(
            needs_layout_passes=False, use_tc_tiling_on_sc=True),
        scratch_types=[
            pltpu.VMEM((2 * _K,), jnp.int32),
            pltpu.VMEM((_NSL * _LROW,), jnp.int32),
            pltpu.VMEM((2, _SR, _W), jnp.float32),
            pltpu.VMEM((2 * _K,), jnp.float32),
            pltpu.SMEM((8,), jnp.int32),
            pltpu.SemaphoreType.DMA,
            pltpu.SemaphoreType.DMA,
            pltpu.SemaphoreType.DMA,
        ],
    )
    def k(x_hbm, loc_hbm, out_hbm, loc_v, lists_v, buf_v, outv_v, cnt_s,
          sem0, sem1, semo):
        wid = lax.axis_index("s") * _NC + lax.axis_index("c")
        base_p = wid * _PPW
        sems = (sem0, sem1)
        lane_i = lax.iota(jnp.int32, _L)

        def fire(step, slot):
            j = step // _NSL
            q = step % _NSL
            p = base_p + j
            h = _SR // 2
            for hh in range(2):
                src = x_hbm.at[
                    p // _C, p % _C, pl.ds(q * _SR + hh * h, h), :]
                pltpu.async_copy(
                    src, buf_v.at[slot].at[pl.ds(hh * h, h)], sems[slot])

        pltpu.sync_copy(loc_hbm, loc_v)
        fire(0, 0)
        fire(1, 1)

        for q in range(_NSL):
            def build(i, ptr):
                s = i * _L
                r = loc_v[pl.ds(s, _L)]
                c = loc_v[pl.ds(_K + s, _L)]
                rloc = lax.bitwise_and(r, _SR - 1)
                w = (lax.shift_left(s + lane_i, 16)
                     | lax.shift_left(rloc, 9) | c)
                m = lax.shift_right_logical(r, 7) == q
                plsc.store_compressed(
                    lists_v.at[pl.ds(q * _LROW + ptr, _L)], w, mask=m)
                return ptr + jnp.sum(jnp.where(m, 1, 0))
            cnt_s[q] = lax.fori_loop(0, _K // _L, build, 0)

        def drain(slot):
            pltpu.make_async_copy(
                x_hbm.at[0, 0, pl.ds(0, _SR), :], buf_v.at[slot],
                sems[slot]).wait()

        def extract(step, slot):
            j = step // _NSL
            q = step % _NSL
            p = base_p + j
            nq = cnt_s[q]
            ovec = outv_v.at[pl.ds((j % 2) * _K, _K)]

            @pl.when((q == 0) & (j >= 2))
            def _():
                pltpu.make_async_copy(
                    outv_v.at[pl.ds(0, _K)], out_hbm.at[0, 0, :],
                    semo).wait()

            def unpack(base):
                w = lists_v[pl.ds(q * _LROW + base, _L)]
                kk = lax.shift_right_logical(w, 16)
                rloc = lax.bitwise_and(
                    lax.shift_right_logical(w, 9), _SR - 1)
                cc = lax.bitwise_and(w, 511)
                return kk, rloc, cc

            def ext_body(i, _):
                kk, rloc, cc = unpack(i * _L)
                vals = plsc.load_gather(buf_v.at[slot], [rloc, cc])
                plsc.store_scatter(ovec, [kk], vals)
                return 0
            nfull = lax.shift_right_logical(nq, 4)
            lax.fori_loop(0, nfull, ext_body, 0)

            @pl.when(lax.bitwise_and(nq, _L - 1) != 0)
            def _():
                base = nfull * _L
                kk, rloc, cc = unpack(base)
                m = (base + lane_i) < nq
                vals = plsc.load_gather(buf_v.at[slot], [rloc, cc], mask=m)
                plsc.store_scatter(ovec, [kk], vals, mask=m)

            @pl.when(q == _NSL - 1)
            def _():
                pltpu.async_copy(ovec, out_hbm.at[p // _C, p % _C, :], semo)

        def round_body(r, _):
            s0 = 2 * r

            @pl.when(s0 > 0)
            def _():
                fire(s0 + 1, 1)
            drain(0)
            extract(s0, 0)

            @pl.when(s0 + 2 < _STEPS)
            def _():
                fire(s0 + 2, 0)
            drain(1)
            extract(s0 + 1, 1)
            return 0
        lax.fori_loop(0, _STEPS // 2, round_body, 0)

        pltpu.make_async_copy(
            outv_v.at[pl.ds(0, _K)], out_hbm.at[0, 0, :], semo).wait()
        pltpu.make_async_copy(
            outv_v.at[pl.ds(0, _K)], out_hbm.at[0, 0, :], semo).wait()

    return k(x, locf)


def kernel(x, loc):
    locf = loc.reshape(2 * _K).astype(jnp.int32)
    return _sc_gather(x, locf)


# speedup vs baseline: 2.5417x; 1.0006x over previous
import functools

import jax
import jax.numpy as jnp
from jax import lax
from jax.experimental import pallas as pl
from jax.experimental.pallas import tpu as pltpu
from jax.experimental.pallas import tpu_sc as plsc

_B, _C, _H, _W = 4, 96, 384, 384
_P = _B * _C
_K = 4096
_NC, _NS = 2, 16
_NW = _NC * _NS
_PPW = _P // _NW
_SR = 128
_NSL = _H // _SR
_STEPS = _PPW * _NSL
_LROW = _K + 16
_L = 16


def _sc_gather(x, locf):
    mesh = plsc.VectorSubcoreMesh(
        core_axis_name="c", subcore_axis_name="s",
        num_cores=_NC, num_subcores=_NS)

    @functools.partial(
        pl.kernel,
        out_type=jax.ShapeDtypeStruct((_B, _C, _K), jnp.float32),
        mesh=mesh,
        compiler_params=pltpu.CompilerParams(
            needs_layout_passes=False, use_tc_tiling_on_sc=True),
        scratch_types=[
            pltpu.VMEM((2 * _K,), jnp.int32),
            pltpu.VMEM((_NSL * _LROW,), jnp.int32),
            pltpu.VMEM((2, _SR, _W), jnp.float32),
            pltpu.VMEM((2 * _K,), jnp.float32),
            pltpu.SMEM((8,), jnp.int32),
            pltpu.SemaphoreType.DMA,
            pltpu.SemaphoreType.DMA,
            pltpu.SemaphoreType.DMA,
        ],
    )
    def k(x_hbm, loc_hbm, out_hbm, loc_v, lists_v, buf_v, outv_v, cnt_s,
          sem0, sem1, semo):
        wid = lax.axis_index("s") * _NC + lax.axis_index("c")
        base_p = wid * _PPW
        sems = (sem0, sem1)
        lane_i = lax.iota(jnp.int32, _L)

        def fire(step, slot):
            j = step // _NSL
            q = step % _NSL
            p = base_p + j
            src = x_hbm.at[p // _C, p % _C, pl.ds(q * _SR, _SR), :]
            pltpu.async_copy(src, buf_v.at[slot], sems[slot])

        pltpu.sync_copy(loc_hbm, loc_v)
        fire(0, 0)
        fire(1, 1)

        for q in range(_NSL):
            def build(i, ptr):
                s = i * _L
                r = loc_v[pl.ds(s, _L)]
                c = loc_v[pl.ds(_K + s, _L)]
                rloc = lax.bitwise_and(r, _SR - 1)
                w = (lax.shift_left(s + lane_i, 16)
                     | lax.shift_left(rloc, 9) | c)
                m = lax.shift_right_logical(r, 7) == q
                plsc.store_compressed(
                    lists_v.at[pl.ds(q * _LROW + ptr, _L)], w, mask=m)
                return ptr + jnp.sum(jnp.where(m, 1, 0))
            cnt_s[q] = lax.fori_loop(0, _K // _L, build, 0)

        def drain(slot):
            pltpu.make_async_copy(
                x_hbm.at[0, 0, pl.ds(0, _SR), :], buf_v.at[slot],
                sems[slot]).wait()

        def extract(step, slot):
            j = step // _NSL
            q = step % _NSL
            p = base_p + j
            nq = cnt_s[q]
            ovec = outv_v.at[pl.ds((j % 2) * _K, _K)]

            @pl.when((q == 0) & (j >= 2))
            def _():
                pltpu.make_async_copy(
                    outv_v.at[pl.ds(0, _K)], out_hbm.at[0, 0, :],
                    semo).wait()

            def unpack(base):
                w = lists_v[pl.ds(q * _LROW + base, _L)]
                kk = lax.shift_right_logical(w, 16)
                rloc = lax.bitwise_and(
                    lax.shift_right_logical(w, 9), _SR - 1)
                cc = lax.bitwise_and(w, 511)
                return kk, rloc, cc

            def ext_body(i, _):
                kk, rloc, cc = unpack(i * _L)
                vals = plsc.load_gather(buf_v.at[slot], [rloc, cc])
                plsc.store_scatter(ovec, [kk], vals)
                return 0
            nfull = lax.shift_right_logical(nq, 4)
            lax.fori_loop(0, nfull, ext_body, 0)

            @pl.when(lax.bitwise_and(nq, _L - 1) != 0)
            def _():
                base = nfull * _L
                kk, rloc, cc = unpack(base)
                m = (base + lane_i) < nq
                vals = plsc.load_gather(buf_v.at[slot], [rloc, cc], mask=m)
                plsc.store_scatter(ovec, [kk], vals, mask=m)

            @pl.when(q == _NSL - 1)
            def _():
                pltpu.async_copy(ovec, out_hbm.at[p // _C, p % _C, :], semo)

        def round_body(r, _):
            s0 = 2 * r

            @pl.when(s0 > 0)
            def _():
                fire(s0 + 1, 1)
            drain(0)
            extract(s0, 0)

            @pl.when(s0 + 2 < _STEPS)
            def _():
                fire(s0 + 2, 0)
            drain(1)
            extract(s0 + 1, 1)
            return 0
        lax.fori_loop(0, _STEPS // 2, round_body, 0)

        pltpu.make_async_copy(
            outv_v.at[pl.ds(0, _K)], out_hbm.at[0, 0, :], semo).wait()
        pltpu.make_async_copy(
            outv_v.at[pl.ds(0, _K)], out_hbm.at[0, 0, :], semo).wait()

    return k(x, locf)


def kernel(x, loc):
    locf = loc.reshape(2 * _K).astype(jnp.int32)
    return _sc_gather(x, locf)
